# Initial kernel scaffold; baseline (speedup 1.0000x reference)
import jax
import jax.numpy as jnp
from jax.experimental import pallas as pl


def kernel(xyz, features):
    raise NotImplementedError("write your pallas kernel here")


# speedup vs baseline: 12.6710x; 12.6710x over previous
import functools

import jax
import jax.numpy as jnp
from jax import lax
from jax.experimental import pallas as pl
from jax.experimental.pallas import tpu as pltpu
from jax.experimental.pallas import tpu_sc as plsc

_M = 512
_S = 32
_R2 = 0.2 * 0.2

_NW = 32
_MW = _M // 4


def _iota16():
    return lax.iota(jnp.int32, 16)


def _sc_centroids(xyz4f, cidx):
    B, N4 = xyz4f.shape
    N = N4 // 4
    mesh = plsc.VectorSubcoreMesh(core_axis_name="c", subcore_axis_name="s")

    @functools.partial(
        pl.kernel,
        out_type=jax.ShapeDtypeStruct((B, _MW * 4 * 4), jnp.float32),
        mesh=mesh,
        scratch_types=[
            pltpu.VMEM((N * 4,), jnp.float32),
            pltpu.VMEM((_MW,), jnp.int32),
            pltpu.VMEM((_MW * 4,), jnp.float32),
        ],
        compiler_params=pltpu.CompilerParams(needs_layout_passes=False),
    )
    def k(xyz4_hbm, cidx_hbm, out_hbm, xyz_v, cid_v, nx_v):
        wid = lax.axis_index("s") * 2 + lax.axis_index("c")
        b = wid // 4
        m0 = (wid % 4) * _MW
        pltpu.sync_copy(xyz4_hbm.at[b], xyz_v)
        pltpu.sync_copy(cidx_hbm.at[b, pl.ds(m0, _MW)], cid_v)
        lane = _iota16()
        for v in range(_MW * 4 // 16):
            rid = plsc.load_gather(cid_v, [v * 4 + (lane >> 2)])
            val = plsc.load_gather(xyz_v, [rid * 4 + (lane & 3)])
            nx_v[pl.ds(v * 16, 16)] = val
        pltpu.sync_copy(nx_v, out_hbm.at[b, pl.ds(m0 * 4, _MW * 4)])

    return k(xyz4f, cidx)


_MT = 64


def _bq_body(xyzT_ref, nc_ref, out_ref):
    N = xyzT_ref.shape[2]
    X = xyzT_ref[0]
    C = nc_ref[0]
    dx = C[:, 0:1] - X[0:1, :]
    dy = C[:, 1:2] - X[1:2, :]
    dz = C[:, 2:3] - X[2:3, :]
    dist = dx * dx + dy * dy + dz * dz
    m = (dist <= _R2).astype(jnp.int32)
    c = m
    sh = 1
    while sh < N:
        z = jnp.zeros((c.shape[0], sh), jnp.int32)
        c = c + jnp.concatenate([z, c[:, :N - sh]], axis=1)
        sh *= 2
    cols = [jnp.sum((c <= s).astype(jnp.int32), axis=1, keepdims=True)
            for s in range(_S)]
    res = jnp.concatenate(cols, axis=1)
    first = res[:, 0:1]
    out_ref[0] = jnp.where(res >= N, first, res)


def _tc_ballquery(xyzT, nxyz4):
    B, _, N = xyzT.shape
    return pl.pallas_call(
        _bq_body,
        grid=(B, _M // _MT),
        in_specs=[
            pl.BlockSpec((1, 8, N), lambda b, j: (b, 0, 0)),
            pl.BlockSpec((1, _MT, 4), lambda b, j: (b, j, 0)),
        ],
        out_specs=pl.BlockSpec((1, _MT, _S), lambda b, j: (b, j, 0)),
        out_shape=jax.ShapeDtypeStruct((B, _M, _S), jnp.int32),
    )(xyzT, nxyz4)


def _sc_group(xyz4f, featsf, idx128, nxflat):
    B, N4 = xyz4f.shape
    N = N4 // 4
    CH = 64
    mesh = plsc.VectorSubcoreMesh(core_axis_name="c", subcore_axis_name="s")
    n_chunk = _MW * _S // 128

    @functools.partial(
        pl.kernel,
        out_type=(
            jax.ShapeDtypeStruct((B, _M, _S * 4), jnp.float32),
            jax.ShapeDtypeStruct((B * _M * _S, CH), jnp.float32),
        ),
        mesh=mesh,
        scratch_types=[
            pltpu.VMEM((N * 4,), jnp.float32),
            pltpu.VMEM((n_chunk, 128), jnp.int32),
            pltpu.VMEM((n_chunk, 128), jnp.int32),
            pltpu.VMEM((_MW * 4,), jnp.float32),
            pltpu.VMEM((_MW, _S * 4), jnp.float32),
            pltpu.VMEM((128, CH), jnp.float32),
            pltpu.SemaphoreType.DMA,
        ],
        compiler_params=pltpu.CompilerParams(
            needs_layout_passes=False, use_tc_tiling_on_sc=False),
    )
    def k(xyz4_hbm, feats_hbm, idx_hbm, nx_hbm, gx_hbm, gf_hbm,
          xyz_v, idx_v, gidx_v, nx_v, gx_v, fr_v, sem):
        wid = lax.axis_index("s") * 2 + lax.axis_index("c")
        b = wid // 4
        q = wid % 4
        m0 = q * _MW
        pltpu.sync_copy(xyz4_hbm.at[b], xyz_v)
        pltpu.sync_copy(idx_hbm.at[b, pl.ds(q * n_chunk, n_chunk)], idx_v)
        pltpu.sync_copy(nx_hbm.at[b, pl.ds(m0 * 4, _MW * 4)], nx_v)
        lane = _iota16()

        def mk_gidx(kk, _):
            for v in range(8):
                gidx_v[kk, pl.ds(v * 16, 16)] = (
                    idx_v[kk, pl.ds(v * 16, 16)] + b * N)
            return 0
        lax.fori_loop(0, n_chunk, mk_gidx, 0, unroll=False)

        def do_cent(ci, _):
            cen_id = ci * 4 + (lane & 3)
            cen = plsc.load_gather(nx_v, [cen_id])
            for v in range(_S * 4 // 16):
                flat = ci * _S + v * 4 + (lane >> 2)
                pid = plsc.load_gather(idx_v, [flat >> 7, flat & 127])
                val = plsc.load_gather(xyz_v, [pid * 4 + (lane & 3)])
                gx_v[ci, pl.ds(v * 16, 16)] = val - cen
            return 0
        lax.fori_loop(0, _MW, do_cent, 0, unroll=False)
        pltpu.sync_copy(gx_v, gx_hbm.at[b, pl.ds(m0, _MW)])

        def do_chunk(kk, _):
            pltpu.async_copy(feats_hbm.at[gidx_v.at[kk]], fr_v, sem).wait()
            pltpu.sync_copy(
                fr_v, gf_hbm.at[pl.ds(wid * _MW * _S + kk * 128, 128)])
            return 0
        lax.fori_loop(0, n_chunk, do_chunk, 0, unroll=False)

    return k(xyz4f, featsf, idx128, nxflat)


def kernel(xyz, features):
    B, N, _ = xyz.shape
    CH = features.shape[-1]
    cidx = jax.random.randint(jax.random.key(42), (B, _M), 0, N,
                              dtype=jnp.int32)
    xyz4f = jnp.pad(xyz, ((0, 0), (0, 0), (0, 1))).reshape(B, N * 4)
    xyzT = jnp.pad(xyz.transpose(0, 2, 1), ((0, 0), (0, 5), (0, 0)))
    nxflat = _sc_centroids(xyz4f, cidx)
    nxyz4 = nxflat.reshape(B, _M, 4)
    idx = _tc_ballquery(xyzT, nxyz4)
    idx128 = idx.reshape(B, _M * _S // 128, 128)
    featsf = features.reshape(B * N, CH)
    gxyz_p, gfeat = _sc_group(xyz4f, featsf, idx128, nxflat)
    grouped_xyz = gxyz_p.reshape(B, _M, _S, 4)[..., :3]
    new_points = jnp.concatenate(
        [grouped_xyz, gfeat.reshape(B, _M, _S, CH)], axis=-1)
    return (nxyz4[..., :3], new_points, cidx, grouped_xyz)
